# baseline (speedup 1.0000x reference)
import jax
import jax.numpy as jnp
from jax import lax
from jax.experimental import pallas as pl
from jax.experimental.pallas import tpu as pltpu, tpu_sc as plsc

_N = 4194304
_G = 16
_L = 16

_NC = 2
_NS = 16
_NW = _NC * _NS

_PER_W = _N // _NW
_CHUNK = 28672
_SIZES = [_CHUNK] * 4 + [_PER_W - 4 * _CHUNK]
_OFFS = [sum(_SIZES[:i]) for i in range(len(_SIZES))]
_NCHUNKS = len(_SIZES)
_NBUF = 2

_GATHER_DNUMS = lax.GatherDimensionNumbers(
    offset_dims=(), collapsed_slice_dims=(0,), start_index_map=(0,))


def _vreg_gather(table, idx):
    return lax.gather(table, idx[:, None], _GATHER_DNUMS, (1,),
                      mode=lax.GatherScatterMode.PROMISE_IN_BOUNDS)


def _sc_body(x_hbm, g_hbm, c_hbm, s_hbm, out_hbm,
             x_v, g_v, c_v, s_v, sem_in, sem_out):
    wid = lax.axis_index("s") * _NC + lax.axis_index("c")
    base = wid * _PER_W

    pltpu.sync_copy(c_hbm, c_v)
    pltpu.sync_copy(s_hbm, s_v)
    c_reg = c_v[...]
    a_reg = 1.0 / s_v[...]

    def start_in(ci):
        b = ci % _NBUF
        off = base + _OFFS[ci]
        n = _SIZES[ci]
        hx = pltpu.async_copy(x_hbm.at[pl.ds(off, n)],
                              x_v[b].at[pl.ds(0, n)], sem_in[b])
        hg = pltpu.async_copy(g_hbm.at[pl.ds(off, n)],
                              g_v[b].at[pl.ds(0, n)], sem_in[b])
        return (hx, hg)

    def start_out(ci):
        b = ci % _NBUF
        off = base + _OFFS[ci]
        n = _SIZES[ci]
        return pltpu.async_copy(x_v[b].at[pl.ds(0, n)],
                                out_hbm.at[pl.ds(off, n)], sem_out[b])

    def compute(ci):
        b = ci % _NBUF
        xb, gb = x_v[b], g_v[b]

        @plsc.parallel_loop(0, _SIZES[ci], step=_L, unroll=8)
        def _body(i):
            sl = pl.ds(i, _L)
            gidx = gb[sl] - 1
            c = _vreg_gather(c_reg, gidx)
            a = _vreg_gather(a_reg, gidx)
            xb[sl] = (xb[sl] - c) * a

    in_h = {}
    out_h = {}
    for ci in range(min(_NBUF, _NCHUNKS)):
        in_h[ci] = start_in(ci)
    for ci in range(_NCHUNKS):
        for h in in_h.pop(ci):
            h.wait()
        compute(ci)
        out_h[ci] = start_out(ci)
        if ci + _NBUF < _NCHUNKS:
            out_h.pop(ci).wait()
            in_h[ci + _NBUF] = start_in(ci + _NBUF)
    for ci in sorted(out_h):
        out_h.pop(ci).wait()


@jax.jit
def _standardize(x, group, centers, scales):
    mesh = plsc.VectorSubcoreMesh(core_axis_name="c", subcore_axis_name="s")
    buf = lambda dt: [pltpu.VMEM((_CHUNK,), dt) for _ in range(_NBUF)]
    return pl.kernel(
        _sc_body,
        out_type=jax.ShapeDtypeStruct((_N,), jnp.float32),
        mesh=mesh,
        scratch_types=[
            buf(jnp.float32),
            buf(jnp.int32),
            pltpu.VMEM((_G,), jnp.float32),
            pltpu.VMEM((_G,), jnp.float32),
            [pltpu.SemaphoreType.DMA for _ in range(_NBUF)],
            [pltpu.SemaphoreType.DMA for _ in range(_NBUF)],
        ],
        compiler_params=pltpu.CompilerParams(needs_layout_passes=False),
    )(x, group, centers, scales)


def kernel(x, group, centers, scales):
    return _standardize(x, group, centers, scales)


# speedup vs baseline: 1.0083x; 1.0083x over previous
import jax
import jax.numpy as jnp
from jax import lax
from jax.experimental import pallas as pl
from jax.experimental.pallas import tpu as pltpu, tpu_sc as plsc

_N = 4194304
_G = 16
_L = 16

_NC = 2
_NS = 16
_NW = _NC * _NS

_PER_W = _N // _NW
_CHUNK = 8192
_SIZES = [_CHUNK] * (_PER_W // _CHUNK)
_OFFS = [sum(_SIZES[:i]) for i in range(len(_SIZES))]
_NCHUNKS = len(_SIZES)
_NBUF = 4

_GATHER_DNUMS = lax.GatherDimensionNumbers(
    offset_dims=(), collapsed_slice_dims=(0,), start_index_map=(0,))


def _vreg_gather(table, idx):
    return lax.gather(table, idx[:, None], _GATHER_DNUMS, (1,),
                      mode=lax.GatherScatterMode.PROMISE_IN_BOUNDS)


def _sc_body(x_hbm, g_hbm, c_hbm, s_hbm, out_hbm,
             x_v, g_v, o_v, c_v, s_v, sem_in, sem_out):
    wid = lax.axis_index("s") * _NC + lax.axis_index("c")
    base = wid * _PER_W

    pltpu.sync_copy(c_hbm, c_v)
    pltpu.sync_copy(s_hbm, s_v)
    c_reg = c_v[...]
    a_reg = 1.0 / s_v[...]

    def start_in(ci):
        b = ci % _NBUF
        off = base + _OFFS[ci]
        n = _SIZES[ci]
        hx = pltpu.async_copy(x_hbm.at[pl.ds(off, n)],
                              x_v[b].at[pl.ds(0, n)], sem_in[b])
        hg = pltpu.async_copy(g_hbm.at[pl.ds(off, n)],
                              g_v[b].at[pl.ds(0, n)], sem_in[b])
        return (hx, hg)

    def start_out(ci):
        b = ci % _NBUF
        off = base + _OFFS[ci]
        n = _SIZES[ci]
        return pltpu.async_copy(o_v[b].at[pl.ds(0, n)],
                                out_hbm.at[pl.ds(off, n)], sem_out[b])

    def compute(ci):
        b = ci % _NBUF
        xb, gb, ob = x_v[b], g_v[b], o_v[b]

        @plsc.parallel_loop(0, _SIZES[ci], step=_L, unroll=8)
        def _body(i):
            sl = pl.ds(i, _L)
            gidx = gb[sl] - 1
            c = _vreg_gather(c_reg, gidx)
            a = _vreg_gather(a_reg, gidx)
            ob[sl] = (xb[sl] - c) * a

    in_h = {}
    out_h = {}
    for ci in range(min(_NBUF, _NCHUNKS)):
        in_h[ci] = start_in(ci)
    for ci in range(_NCHUNKS):
        for h in in_h.pop(ci):
            h.wait()
        if ci - _NBUF in out_h:
            out_h.pop(ci - _NBUF).wait()
        compute(ci)
        out_h[ci] = start_out(ci)
        if ci + _NBUF < _NCHUNKS:
            in_h[ci + _NBUF] = start_in(ci + _NBUF)
    for ci in sorted(out_h):
        out_h.pop(ci).wait()


@jax.jit
def _standardize(x, group, centers, scales):
    mesh = plsc.VectorSubcoreMesh(core_axis_name="c", subcore_axis_name="s")
    buf = lambda dt: [pltpu.VMEM((_CHUNK,), dt) for _ in range(_NBUF)]
    return pl.kernel(
        _sc_body,
        out_type=jax.ShapeDtypeStruct((_N,), jnp.float32),
        mesh=mesh,
        scratch_types=[
            buf(jnp.float32),
            buf(jnp.int32),
            buf(jnp.float32),
            pltpu.VMEM((_G,), jnp.float32),
            pltpu.VMEM((_G,), jnp.float32),
            [pltpu.SemaphoreType.DMA for _ in range(_NBUF)],
            [pltpu.SemaphoreType.DMA for _ in range(_NBUF)],
        ],
        compiler_params=pltpu.CompilerParams(needs_layout_passes=False),
    )(x, group, centers, scales)


def kernel(x, group, centers, scales):
    return _standardize(x, group, centers, scales)
